# Initial kernel scaffold; baseline (speedup 1.0000x reference)
import jax
import jax.numpy as jnp
from jax.experimental import pallas as pl


def kernel(x, edge_index, batch, Wp, bp, W1, b1, W2, b2, Ws, bs, prelu_w):
    raise NotImplementedError("write your pallas kernel here")


# speedup vs baseline: 2.3642x; 2.3642x over previous
import functools

import jax
import jax.numpy as jnp
from jax import lax
from jax.experimental import pallas as pl
from jax.experimental.pallas import tpu as pltpu
from jax.experimental.pallas import tpu_sc as plsc

N = 10000
E = 320000
D_IN = 128
D_H = 300
D_OUT = 1024
DEPTH = 5
NG = 128

PW = 128
P = 3
DP = P * PW
NSUB = 16

CH = 80
NCH1 = (E // NSUB) // CH
NCH2 = (E // 2 // NSUB) // CH

RB = 400
NRB = N // RB

BLK = 1000


_MESH = plsc.VectorSubcoreMesh(core_axis_name="c", subcore_axis_name="s")


@functools.partial(
    pl.kernel,
    out_type=jax.ShapeDtypeStruct((4 * N, PW), jnp.float32),
    mesh=_MESH,
    scratch_types=[
        pltpu.VMEM((CH,), jnp.int32),
        pltpu.VMEM((CH,), jnp.int32),
        pltpu.VMEM((CH, PW), jnp.float32),
        pltpu.VMEM_SHARED((N, PW), jnp.float32),
        pltpu.SemaphoreType.DMA,
    ],
)
def _sc_agg(h_hbm, src_hbm, dst_hbm, out_hbm, srcb, dstb, rows, agg, sem):
    c = lax.axis_index("c")
    s = lax.axis_index("s")

    def init_from(piece_base):
        for k in range(2):
            j = s + NSUB * k

            @pl.when(j < NRB)
            def _():
                pltpu.sync_copy(h_hbm.at[pl.ds(piece_base + j * RB, RB)],
                                agg.at[pl.ds(j * RB, RB)])

    def writeout(out_base):
        for k in range(2):
            j = s + NSUB * k

            @pl.when(j < NRB)
            def _():
                pltpu.sync_copy(agg.at[pl.ds(j * RB, RB)],
                                out_hbm.at[pl.ds(out_base + j * RB, RB)])

    def edge_pass(nch, edge_base, piece_base):
        def chunk(k, carry):
            st = edge_base + k * CH
            pltpu.sync_copy(src_hbm.at[pl.ds(st, CH)], srcb)
            pltpu.sync_copy(dst_hbm.at[pl.ds(st, CH)], dstb)
            for q in range(CH // 16):
                srcb[pl.ds(q * 16, 16)] = srcb[pl.ds(q * 16, 16)] + piece_base
            pltpu.async_copy(h_hbm.at[srcb], rows, sem).wait()
            pltpu.sync_copy(rows, agg.at[dstb], add=True)
            return carry

        lax.fori_loop(0, nch, chunk, 0)

    init_from(c * N)
    plsc.subcore_barrier()
    edge_pass(NCH1, s * (E // NSUB), c * N)
    plsc.subcore_barrier()
    writeout(c * N)
    plsc.subcore_barrier()

    init_from(2 * N)
    plsc.subcore_barrier()
    edge_pass(NCH2, c * (E // 2) + s * (E // 2 // NSUB), 2 * N)
    plsc.subcore_barrier()
    writeout((2 + c) * N)


def _split_store(o_ref, v):
    for p in range(P):
        o_ref[p] = v[:, p * PW:(p + 1) * PW]


def _proj_body(x_ref, w_ref, b_ref, o_ref):
    h = jnp.dot(x_ref[...], w_ref[...], preferred_element_type=jnp.float32)
    _split_store(o_ref, jnp.maximum(h + b_ref[...], 0.0))


def _layer_body(z_ref, h2_ref, w1_ref, b1_ref, w2_ref, b2_ref, o_ref, *,
                last):
    z = jnp.concatenate(
        [z_ref[0], z_ref[1], z_ref[2] + z_ref[3] - h2_ref[0]], axis=1)
    a = jnp.dot(z, w1_ref[...], preferred_element_type=jnp.float32)
    a = jnp.maximum(a + b1_ref[...], 0.0)
    o = jnp.dot(a, w2_ref[...], preferred_element_type=jnp.float32)
    o = o + b2_ref[...]
    if not last:
        o = jnp.maximum(o, 0.0)
    _split_store(o_ref, o)


def _readout_body(h_ref, batch_ref, ws_ref, bs_ref, pw_ref, o_ref, acc_ref):
    i = pl.program_id(0)

    @pl.when(i == 0)
    def _():
        acc_ref[...] = jnp.zeros((NG, DP), jnp.float32)

    z = jnp.concatenate([h_ref[p] for p in range(P)], axis=1)
    ids = lax.broadcasted_iota(jnp.int32, (NG, BLK), 0)
    sel = (jnp.broadcast_to(batch_ref[0], (NG, BLK)) == ids)
    acc_ref[...] += jnp.dot(sel.astype(jnp.float32), z,
                            preferred_element_type=jnp.float32)

    @pl.when(i == pl.num_programs(0) - 1)
    def _():
        r = jnp.dot(acc_ref[...], ws_ref[...],
                    preferred_element_type=jnp.float32) + bs_ref[...]
        o_ref[...] = jnp.where(r >= 0.0, r, r * pw_ref[...])


def _proj(x, wp, bp):
    return pl.pallas_call(
        _proj_body,
        grid=(N // BLK,),
        in_specs=[
            pl.BlockSpec((BLK, D_IN), lambda i: (i, 0)),
            pl.BlockSpec((D_IN, DP), lambda i: (0, 0)),
            pl.BlockSpec((1, DP), lambda i: (0, 0)),
        ],
        out_specs=pl.BlockSpec((P, BLK, PW), lambda i: (0, i, 0)),
        out_shape=jax.ShapeDtypeStruct((P, N, PW), jnp.float32),
    )(x, wp, bp)


def _layer(z4, h3, w1, b1, w2, b2, last):
    return pl.pallas_call(
        functools.partial(_layer_body, last=last),
        grid=(N // BLK,),
        in_specs=[
            pl.BlockSpec((4, BLK, PW), lambda i: (0, i, 0)),
            pl.BlockSpec((1, BLK, PW), lambda i: (2, i, 0)),
            pl.BlockSpec((DP, DP), lambda i: (0, 0)),
            pl.BlockSpec((1, DP), lambda i: (0, 0)),
            pl.BlockSpec((DP, DP), lambda i: (0, 0)),
            pl.BlockSpec((1, DP), lambda i: (0, 0)),
        ],
        out_specs=pl.BlockSpec((P, BLK, PW), lambda i: (0, i, 0)),
        out_shape=jax.ShapeDtypeStruct((P, N, PW), jnp.float32),
    )(z4, h3, w1, b1, w2, b2)


def _readout(h3, batch, ws, bs, pw):
    return pl.pallas_call(
        _readout_body,
        grid=(N // BLK,),
        in_specs=[
            pl.BlockSpec((P, BLK, PW), lambda i: (0, i, 0)),
            pl.BlockSpec((1, 1, BLK), lambda i: (i, 0, 0)),
            pl.BlockSpec((DP, D_OUT), lambda i: (0, 0)),
            pl.BlockSpec((1, D_OUT), lambda i: (0, 0)),
            pl.BlockSpec((1, 1), lambda i: (0, 0)),
        ],
        out_specs=pl.BlockSpec((NG, D_OUT), lambda i: (0, 0)),
        out_shape=jax.ShapeDtypeStruct((NG, D_OUT), jnp.float32),
        scratch_shapes=[pltpu.VMEM((NG, DP), jnp.float32)],
    )(h3, batch, ws, bs, pw)


def kernel(x, edge_index, batch, Wp, bp, W1, b1, W2, b2, Ws, bs, prelu_w):
    src = edge_index[0]
    dst = edge_index[1]

    pad = DP - D_H
    wp_p = jnp.pad(Wp, ((0, 0), (0, pad)))
    bp_p = jnp.pad(bp, (0, pad)).reshape(1, DP)
    w1_p = jnp.pad(W1, ((0, 0), (0, pad), (0, pad)))
    b1_p = jnp.pad(b1, ((0, 0), (0, pad))).reshape(DEPTH, 1, DP)
    w2_p = jnp.pad(W2, ((0, 0), (0, pad), (0, pad)))
    b2_p = jnp.pad(b2, ((0, 0), (0, pad))).reshape(DEPTH, 1, DP)
    ws_p = jnp.pad(Ws, ((0, pad), (0, 0)))
    bs_p = bs.reshape(1, D_OUT)
    pw = prelu_w.reshape(1, 1)
    batch2 = batch.reshape(N // BLK, 1, BLK)

    h = _proj(x, wp_p, bp_p)
    for i in range(DEPTH):
        z = _sc_agg(h.reshape(P * N, PW), src, dst)
        h = _layer(z.reshape(4, N, PW), h, w1_p[i], b1_p[i], w2_p[i],
                   b2_p[i], last=(i == DEPTH - 1))
    return _readout(h, batch2, ws_p, bs_p, pw)


# speedup vs baseline: 4.8712x; 2.0604x over previous
import functools

import jax
import jax.numpy as jnp
from jax import lax
from jax.experimental import pallas as pl
from jax.experimental.pallas import tpu as pltpu
from jax.experimental.pallas import tpu_sc as plsc

N = 10000
E = 320000
D_IN = 128
D_H = 300
D_OUT = 1024
DEPTH = 5
NG = 128

PW = 128
P = 3
DP = P * PW
NSUB = 16

ECH = 128
ER = E // ECH
ERH = ER // 2
RB = 400
NRB = N // RB

BLK = 1000


_MESH = plsc.VectorSubcoreMesh(core_axis_name="c", subcore_axis_name="s")


@functools.partial(
    pl.kernel,
    out_type=jax.ShapeDtypeStruct((4 * N, PW), jnp.float32),
    mesh=_MESH,
    scratch_types=[
        pltpu.VMEM((ECH,), jnp.int32),
        pltpu.VMEM((ECH,), jnp.int32),
        pltpu.VMEM((ECH,), jnp.int32),
        pltpu.VMEM((ECH,), jnp.int32),
        pltpu.VMEM((ECH, PW), jnp.float32),
        pltpu.VMEM((ECH, PW), jnp.float32),
        pltpu.VMEM_SHARED((N, PW), jnp.float32),
        pltpu.SemaphoreType.DMA,
        pltpu.SemaphoreType.DMA,
    ],
)
def _sc_agg(h_hbm, src3_hbm, dst_hbm, out_hbm, scur0, scur1, dcur0, dcur1,
            rows0, rows1, agg, gs0, gs1):
    c = lax.axis_index("c")
    s = lax.axis_index("s")
    scur = (scur0, scur1)
    dcur = (dcur0, dcur1)
    rows = (rows0, rows1)
    gsem = (gs0, gs1)

    def init_from(piece_base):
        for k in range(2):
            j = s + NSUB * k

            @pl.when(j < NRB)
            def _():
                pltpu.sync_copy(h_hbm.at[pl.ds(piece_base + j * RB, RB)],
                                agg.at[pl.ds(j * RB, RB)])

    def writeout(out_base):
        for k in range(2):
            j = s + NSUB * k

            @pl.when(j < NRB)
            def _():
                pltpu.sync_copy(agg.at[pl.ds(j * RB, RB)],
                                out_hbm.at[pl.ds(out_base + j * RB, RB)])

    def edge_pipeline(cnt, src_base, dst_base):
        def load_idx(k, p):
            soff = pl.multiple_of(src_base + k * ECH, 8)
            doff = pl.multiple_of(dst_base + k * ECH, 8)
            pltpu.sync_copy(src3_hbm.at[pl.ds(soff, ECH)], scur[p])
            pltpu.sync_copy(dst_hbm.at[pl.ds(doff, ECH)], dcur[p])

        load_idx(0, 0)
        pltpu.async_copy(h_hbm.at[scur0], rows0, gs0)

        def step(k, p, q):
            @pl.when(k + 1 < cnt)
            def _():
                load_idx(k + 1, q)

            pltpu.make_async_copy(h_hbm.at[scur[p]], rows[p],
                                  gsem[p]).wait()

            @pl.when(k + 1 < cnt)
            def _():
                pltpu.async_copy(h_hbm.at[scur[q]], rows[q], gsem[q])

            pltpu.sync_copy(rows[p], agg.at[dcur[p]], add=True)

        def pair(k2, carry):
            k = k2 * 2
            step(k, 0, 1)

            @pl.when(k + 1 < cnt)
            def _():
                step(k + 1, 1, 0)

            return carry

        lax.fori_loop(0, (cnt + 1) // 2, pair, 0)

    def bounds(total, sub):
        start = (sub * total) // NSUB
        return start, ((sub + 1) * total) // NSUB - start

    init_from(c * N)
    r0, cnt1 = bounds(ER, s)
    plsc.subcore_barrier()
    edge_pipeline(cnt1, c * E + r0 * ECH, r0 * ECH)
    plsc.subcore_barrier()
    writeout(c * N)
    plsc.subcore_barrier()

    init_from(2 * N)
    r2, cnt2 = bounds(ERH, s)
    g2 = c * ERH + r2
    plsc.subcore_barrier()
    edge_pipeline(cnt2, 2 * E + g2 * ECH, g2 * ECH)
    plsc.subcore_barrier()
    writeout((2 + c) * N)


def _shift_body(s_ref, o_ref):
    for p in range(P):
        o_ref[p] = s_ref[0] + p * N


def _split_store(o_ref, v):
    for p in range(P):
        o_ref[p] = v[:, p * PW:(p + 1) * PW]


def _proj_body(x_ref, w_ref, b_ref, o_ref):
    h = jnp.dot(x_ref[...], w_ref[...], preferred_element_type=jnp.float32)
    _split_store(o_ref, jnp.maximum(h + b_ref[...], 0.0))


def _layer_body(z_ref, h2_ref, w1_ref, b1_ref, w2_ref, b2_ref, o_ref, *,
                last):
    z = jnp.concatenate(
        [z_ref[0], z_ref[1], z_ref[2] + z_ref[3] - h2_ref[0]], axis=1)
    a = jnp.dot(z, w1_ref[...], preferred_element_type=jnp.float32)
    a = jnp.maximum(a + b1_ref[...], 0.0)
    o = jnp.dot(a, w2_ref[...], preferred_element_type=jnp.float32)
    o = o + b2_ref[...]
    if not last:
        o = jnp.maximum(o, 0.0)
    _split_store(o_ref, o)


def _readout_body(h_ref, batch_ref, ws_ref, bs_ref, pw_ref, o_ref, acc_ref):
    i = pl.program_id(0)

    @pl.when(i == 0)
    def _():
        acc_ref[...] = jnp.zeros((NG, DP), jnp.float32)

    z = jnp.concatenate([h_ref[p] for p in range(P)], axis=1)
    ids = lax.broadcasted_iota(jnp.int32, (NG, BLK), 0)
    sel = (jnp.broadcast_to(batch_ref[0], (NG, BLK)) == ids)
    acc_ref[...] += jnp.dot(sel.astype(jnp.float32), z,
                            preferred_element_type=jnp.float32)

    @pl.when(i == pl.num_programs(0) - 1)
    def _():
        r = jnp.dot(acc_ref[...], ws_ref[...],
                    preferred_element_type=jnp.float32) + bs_ref[...]
        o_ref[...] = jnp.where(r >= 0.0, r, r * pw_ref[...])


def _shift_src(src):
    return pl.pallas_call(
        _shift_body,
        grid=(5,),
        in_specs=[pl.BlockSpec((1, E // 5), lambda i: (0, i))],
        out_specs=pl.BlockSpec((P, E // 5), lambda i: (0, i)),
        out_shape=jax.ShapeDtypeStruct((P, E), jnp.int32),
    )(src.reshape(1, E))


def _proj(x, wp, bp):
    return pl.pallas_call(
        _proj_body,
        grid=(N // BLK,),
        in_specs=[
            pl.BlockSpec((BLK, D_IN), lambda i: (i, 0)),
            pl.BlockSpec((D_IN, DP), lambda i: (0, 0)),
            pl.BlockSpec((1, DP), lambda i: (0, 0)),
        ],
        out_specs=pl.BlockSpec((P, BLK, PW), lambda i: (0, i, 0)),
        out_shape=jax.ShapeDtypeStruct((P, N, PW), jnp.float32),
    )(x, wp, bp)


def _layer(z4, h3, w1, b1, w2, b2, last):
    return pl.pallas_call(
        functools.partial(_layer_body, last=last),
        grid=(N // BLK,),
        in_specs=[
            pl.BlockSpec((4, BLK, PW), lambda i: (0, i, 0)),
            pl.BlockSpec((1, BLK, PW), lambda i: (2, i, 0)),
            pl.BlockSpec((DP, DP), lambda i: (0, 0)),
            pl.BlockSpec((1, DP), lambda i: (0, 0)),
            pl.BlockSpec((DP, DP), lambda i: (0, 0)),
            pl.BlockSpec((1, DP), lambda i: (0, 0)),
        ],
        out_specs=pl.BlockSpec((P, BLK, PW), lambda i: (0, i, 0)),
        out_shape=jax.ShapeDtypeStruct((P, N, PW), jnp.float32),
    )(z4, h3, w1, b1, w2, b2)


def _readout(h3, batch, ws, bs, pw):
    return pl.pallas_call(
        _readout_body,
        grid=(N // BLK,),
        in_specs=[
            pl.BlockSpec((P, BLK, PW), lambda i: (0, i, 0)),
            pl.BlockSpec((1, 1, BLK), lambda i: (i, 0, 0)),
            pl.BlockSpec((DP, D_OUT), lambda i: (0, 0)),
            pl.BlockSpec((1, D_OUT), lambda i: (0, 0)),
            pl.BlockSpec((1, 1), lambda i: (0, 0)),
        ],
        out_specs=pl.BlockSpec((NG, D_OUT), lambda i: (0, 0)),
        out_shape=jax.ShapeDtypeStruct((NG, D_OUT), jnp.float32),
        scratch_shapes=[pltpu.VMEM((NG, DP), jnp.float32)],
    )(h3, batch, ws, bs, pw)


def kernel(x, edge_index, batch, Wp, bp, W1, b1, W2, b2, Ws, bs, prelu_w):
    src = edge_index[0]
    dst = edge_index[1]

    pad = DP - D_H
    wp_p = jnp.pad(Wp, ((0, 0), (0, pad)))
    bp_p = jnp.pad(bp, (0, pad)).reshape(1, DP)
    w1_p = jnp.pad(W1, ((0, 0), (0, pad), (0, pad)))
    b1_p = jnp.pad(b1, ((0, 0), (0, pad))).reshape(DEPTH, 1, DP)
    w2_p = jnp.pad(W2, ((0, 0), (0, pad), (0, pad)))
    b2_p = jnp.pad(b2, ((0, 0), (0, pad))).reshape(DEPTH, 1, DP)
    ws_p = jnp.pad(Ws, ((0, pad), (0, 0)))
    bs_p = bs.reshape(1, D_OUT)
    pw = prelu_w.reshape(1, 1)
    batch2 = batch.reshape(N // BLK, 1, BLK)

    src3 = jnp.pad(_shift_src(src).reshape(P * E), (0, 8 * ECH))
    dst2 = jnp.pad(dst, (0, 8 * ECH))

    h = _proj(x, wp_p, bp_p)
    for i in range(DEPTH):
        z = _sc_agg(h.reshape(P * N, PW), src3, dst2)
        h = _layer(z.reshape(4, N, PW), h, w1_p[i], b1_p[i], w2_p[i],
                   b2_p[i], last=(i == DEPTH - 1))
    return _readout(h, batch2, ws_p, bs_p, pw)


# speedup vs baseline: 5.3723x; 1.1029x over previous
import functools

import jax
import jax.numpy as jnp
from jax import lax
from jax.experimental import pallas as pl
from jax.experimental.pallas import tpu as pltpu
from jax.experimental.pallas import tpu_sc as plsc

N = 10000
E = 320000
D_IN = 128
D_H = 300
D_OUT = 1024
DEPTH = 5
NG = 128

PW = 128
P = 3
DP = P * PW
NSUB = 16

ECH = 128
ER = E // ECH
ERH = ER // 2
MB = 8
PAIR = 2 * MB

RB = 400
NRB = N // RB

BLK = 1000


_MESH = plsc.VectorSubcoreMesh(core_axis_name="c", subcore_axis_name="s")


@functools.partial(
    pl.kernel,
    out_type=jax.ShapeDtypeStruct((4 * N, PW), jnp.float32),
    mesh=_MESH,
    scratch_types=[
        pltpu.VMEM((MB * ECH,), jnp.int32),
        pltpu.VMEM((MB * ECH,), jnp.int32),
        pltpu.VMEM((MB * ECH,), jnp.int32),
        pltpu.VMEM((MB * ECH,), jnp.int32),
        pltpu.VMEM((ECH,), jnp.int32),
        pltpu.VMEM((ECH,), jnp.int32),
        pltpu.VMEM((ECH,), jnp.int32),
        pltpu.VMEM((ECH,), jnp.int32),
        pltpu.VMEM((ECH, PW), jnp.float32),
        pltpu.VMEM((ECH, PW), jnp.float32),
        pltpu.VMEM_SHARED((N, PW), jnp.float32),
        pltpu.SemaphoreType.DMA,
        pltpu.SemaphoreType.DMA,
        pltpu.SemaphoreType.DMA,
    ],
)
def _sc_agg(h_hbm, src3_hbm, dst_hbm, out_hbm, sbufA, sbufB, dbufA, dbufB,
            scur0, scur1, dcur0, dcur1, rows0, rows1, agg, gs0, gs1, isem):
    c = lax.axis_index("c")
    s = lax.axis_index("s")
    sbuf = (sbufA, sbufB)
    dbuf = (dbufA, dbufB)
    scur = (scur0, scur1)
    dcur = (dcur0, dcur1)
    rows = (rows0, rows1)
    gsem = (gs0, gs1)

    def init_from(piece_base):
        for k in range(2):
            j = s + NSUB * k

            @pl.when(j < NRB)
            def _():
                pltpu.sync_copy(h_hbm.at[pl.ds(piece_base + j * RB, RB)],
                                agg.at[pl.ds(j * RB, RB)])

    def writeout(out_base):
        for k in range(2):
            j = s + NSUB * k

            @pl.when(j < NRB)
            def _():
                pltpu.sync_copy(agg.at[pl.ds(j * RB, RB)],
                                out_hbm.at[pl.ds(out_base + j * RB, RB)])

    def edge_pipeline(cnt, src_base, dst_base):
        def macro_refs(m, ab):
            soff = pl.multiple_of(src_base + m * MB * ECH, 8)
            doff = pl.multiple_of(dst_base + m * MB * ECH, 8)
            return (src3_hbm.at[pl.ds(soff, MB * ECH)], sbuf[ab],
                    dst_hbm.at[pl.ds(doff, MB * ECH)], dbuf[ab])

        def idx_load(m, ab):
            sref, sdst, dref, ddst = macro_refs(m, ab)
            pltpu.async_copy(sref, sdst, isem)
            pltpu.async_copy(dref, ddst, isem)

        def idx_wait(m, ab):
            sref, sdst, dref, ddst = macro_refs(m, ab)
            pltpu.make_async_copy(sref, sdst, isem).wait()
            pltpu.make_async_copy(dref, ddst, isem).wait()

        def fill(cur, buf, jm):
            for t in range(ECH // 16):
                cur[pl.ds(t * 16, 16)] = buf[pl.ds(jm * ECH + t * 16, 16)]

        idx_load(0, 0)

        @pl.when(MB < cnt)
        def _():
            idx_load(1, 1)

        def body(m2, carry):
            kbase = m2 * PAIR
            idx_wait(2 * m2, 0)

            @pl.when(kbase + MB < cnt)
            def _():
                idx_wait(2 * m2 + 1, 1)

            fill(scur0, sbufA, 0)
            pltpu.async_copy(h_hbm.at[scur0], rows0, gs0)

            for j in range(PAIR):
                k = kbase + j
                p, q = j % 2, 1 - j % 2
                ab = j // MB

                @pl.when(k < cnt)
                def _(j=j, k=k, p=p, q=q, ab=ab):
                    if j < PAIR - 1:
                        @pl.when(k + 1 < cnt)
                        def _():
                            fill(scur[q], sbuf[(j + 1) // MB],
                                 (j + 1) % MB)

                    pltpu.make_async_copy(h_hbm.at[scur[p]], rows[p],
                                          gsem[p]).wait()

                    if j < PAIR - 1:
                        @pl.when(k + 1 < cnt)
                        def _():
                            pltpu.async_copy(h_hbm.at[scur[q]], rows[q],
                                             gsem[q])

                    fill(dcur[p], dbuf[ab], j % MB)
                    pltpu.sync_copy(rows[p], agg.at[dcur[p]], add=True)

                if j == MB - 1:
                    @pl.when(kbase + PAIR < cnt)
                    def _(m2=m2):
                        idx_load(2 * m2 + 2, 0)

                if j == PAIR - 1:
                    @pl.when(kbase + PAIR + MB < cnt)
                    def _(m2=m2):
                        idx_load(2 * m2 + 3, 1)

            return carry

        lax.fori_loop(0, (cnt + PAIR - 1) // PAIR, body, 0)

    def bounds(total, sub):
        start = (sub * total) // NSUB
        return start, ((sub + 1) * total) // NSUB - start

    init_from(c * N)
    r0, cnt1 = bounds(ER, s)
    plsc.subcore_barrier()
    edge_pipeline(cnt1, c * E + r0 * ECH, r0 * ECH)
    plsc.subcore_barrier()
    writeout(c * N)
    plsc.subcore_barrier()

    init_from(2 * N)
    r2, cnt2 = bounds(ERH, s)
    g2 = c * ERH + r2
    plsc.subcore_barrier()
    edge_pipeline(cnt2, 2 * E + g2 * ECH, g2 * ECH)
    plsc.subcore_barrier()
    writeout((2 + c) * N)


def _shift_body(s_ref, o_ref):
    for p in range(P):
        o_ref[p] = s_ref[0] + p * N


def _split_store(o_ref, v):
    for p in range(P):
        o_ref[p] = v[:, p * PW:(p + 1) * PW]


def _proj_body(x_ref, w_ref, b_ref, o_ref):
    h = jnp.dot(x_ref[...], w_ref[...], preferred_element_type=jnp.float32)
    _split_store(o_ref, jnp.maximum(h + b_ref[...], 0.0))


def _layer_body(z_ref, h2_ref, w1_ref, b1_ref, w2_ref, b2_ref, o_ref, *,
                last):
    z = jnp.concatenate(
        [z_ref[0], z_ref[1], z_ref[2] + z_ref[3] - h2_ref[0]], axis=1)
    a = jnp.dot(z, w1_ref[...], preferred_element_type=jnp.float32)
    a = jnp.maximum(a + b1_ref[...], 0.0)
    o = jnp.dot(a, w2_ref[...], preferred_element_type=jnp.float32)
    o = o + b2_ref[...]
    if not last:
        o = jnp.maximum(o, 0.0)
    _split_store(o_ref, o)


def _readout_body(h_ref, batch_ref, ws_ref, bs_ref, pw_ref, o_ref, acc_ref):
    i = pl.program_id(0)

    @pl.when(i == 0)
    def _():
        acc_ref[...] = jnp.zeros((NG, DP), jnp.float32)

    z = jnp.concatenate([h_ref[p] for p in range(P)], axis=1)
    ids = lax.broadcasted_iota(jnp.int32, (NG, BLK), 0)
    sel = (jnp.broadcast_to(batch_ref[0], (NG, BLK)) == ids)
    acc_ref[...] += jnp.dot(sel.astype(jnp.float32), z,
                            preferred_element_type=jnp.float32)

    @pl.when(i == pl.num_programs(0) - 1)
    def _():
        r = jnp.dot(acc_ref[...], ws_ref[...],
                    preferred_element_type=jnp.float32) + bs_ref[...]
        o_ref[...] = jnp.where(r >= 0.0, r, r * pw_ref[...])


def _shift_src(src):
    return pl.pallas_call(
        _shift_body,
        grid=(5,),
        in_specs=[pl.BlockSpec((1, E // 5), lambda i: (0, i))],
        out_specs=pl.BlockSpec((P, E // 5), lambda i: (0, i)),
        out_shape=jax.ShapeDtypeStruct((P, E), jnp.int32),
    )(src.reshape(1, E))


def _proj(x, wp, bp):
    return pl.pallas_call(
        _proj_body,
        grid=(N // BLK,),
        in_specs=[
            pl.BlockSpec((BLK, D_IN), lambda i: (i, 0)),
            pl.BlockSpec((D_IN, DP), lambda i: (0, 0)),
            pl.BlockSpec((1, DP), lambda i: (0, 0)),
        ],
        out_specs=pl.BlockSpec((P, BLK, PW), lambda i: (0, i, 0)),
        out_shape=jax.ShapeDtypeStruct((P, N, PW), jnp.float32),
    )(x, wp, bp)


def _layer(z4, h3, w1, b1, w2, b2, last):
    return pl.pallas_call(
        functools.partial(_layer_body, last=last),
        grid=(N // BLK,),
        in_specs=[
            pl.BlockSpec((4, BLK, PW), lambda i: (0, i, 0)),
            pl.BlockSpec((1, BLK, PW), lambda i: (2, i, 0)),
            pl.BlockSpec((DP, DP), lambda i: (0, 0)),
            pl.BlockSpec((1, DP), lambda i: (0, 0)),
            pl.BlockSpec((DP, DP), lambda i: (0, 0)),
            pl.BlockSpec((1, DP), lambda i: (0, 0)),
        ],
        out_specs=pl.BlockSpec((P, BLK, PW), lambda i: (0, i, 0)),
        out_shape=jax.ShapeDtypeStruct((P, N, PW), jnp.float32),
    )(z4, h3, w1, b1, w2, b2)


def _readout(h3, batch, ws, bs, pw):
    return pl.pallas_call(
        _readout_body,
        grid=(N // BLK,),
        in_specs=[
            pl.BlockSpec((P, BLK, PW), lambda i: (0, i, 0)),
            pl.BlockSpec((1, 1, BLK), lambda i: (i, 0, 0)),
            pl.BlockSpec((DP, D_OUT), lambda i: (0, 0)),
            pl.BlockSpec((1, D_OUT), lambda i: (0, 0)),
            pl.BlockSpec((1, 1), lambda i: (0, 0)),
        ],
        out_specs=pl.BlockSpec((NG, D_OUT), lambda i: (0, 0)),
        out_shape=jax.ShapeDtypeStruct((NG, D_OUT), jnp.float32),
        scratch_shapes=[pltpu.VMEM((NG, DP), jnp.float32)],
    )(h3, batch, ws, bs, pw)


def kernel(x, edge_index, batch, Wp, bp, W1, b1, W2, b2, Ws, bs, prelu_w):
    src = edge_index[0]
    dst = edge_index[1]

    pad = DP - D_H
    wp_p = jnp.pad(Wp, ((0, 0), (0, pad)))
    bp_p = jnp.pad(bp, (0, pad)).reshape(1, DP)
    w1_p = jnp.pad(W1, ((0, 0), (0, pad), (0, pad)))
    b1_p = jnp.pad(b1, ((0, 0), (0, pad))).reshape(DEPTH, 1, DP)
    w2_p = jnp.pad(W2, ((0, 0), (0, pad), (0, pad)))
    b2_p = jnp.pad(b2, ((0, 0), (0, pad))).reshape(DEPTH, 1, DP)
    ws_p = jnp.pad(Ws, ((0, pad), (0, 0)))
    bs_p = bs.reshape(1, D_OUT)
    pw = prelu_w.reshape(1, 1)
    batch2 = batch.reshape(N // BLK, 1, BLK)

    src3 = jnp.pad(_shift_src(src).reshape(P * E), (0, 8 * ECH))
    dst2 = jnp.pad(dst, (0, 8 * ECH))

    h = _proj(x, wp_p, bp_p)
    for i in range(DEPTH):
        z = _sc_agg(h.reshape(P * N, PW), src3, dst2)
        h = _layer(z.reshape(4, N, PW), h, w1_p[i], b1_p[i], w2_p[i],
                   b2_p[i], last=(i == DEPTH - 1))
    return _readout(h, batch2, ws_p, bs_p, pw)
